# baseline (speedup 1.0000x reference)
import jax
import jax.numpy as jnp
from jax.experimental import pallas as pl

N = 1048576
IN_CH = 9
OUT_CH = 64
BN_EPS = 1e-3
G = N // 8
ROWSG1 = 4096
ROWSG2 = 2048


@jax.jit
def kernel(features, W, gamma, beta):
    x3 = features.reshape(G, 8, IN_CH)
    wb = W.astype(jnp.bfloat16)
    g2 = gamma.reshape(1, OUT_CH)
    b2 = beta.reshape(1, OUT_CH)

    def stats_body(x_ref, w_ref, o_ref):
        i = pl.program_id(0)
        xb = x_ref[...].reshape(ROWSG1 * 8, IN_CH).astype(jnp.bfloat16)
        h = jnp.dot(xb, w_ref[...], preferred_element_type=jnp.float32)
        s = jnp.sum(h, axis=0, keepdims=True)
        q = jnp.sum(h * h, axis=0, keepdims=True)
        blk = jnp.concatenate([s, q], axis=0)

        @pl.when(i == 0)
        def _init():
            o_ref[...] = blk

        @pl.when(i > 0)
        def _acc():
            o_ref[...] = o_ref[...] + blk

    stats = pl.pallas_call(
        stats_body,
        grid=(G // ROWSG1,),
        in_specs=[
            pl.BlockSpec((ROWSG1, 8, IN_CH), lambda i: (i, 0, 0)),
            pl.BlockSpec((IN_CH, OUT_CH), lambda i: (0, 0)),
        ],
        out_specs=pl.BlockSpec((2, OUT_CH), lambda i: (0, 0)),
        out_shape=jax.ShapeDtypeStruct((2, OUT_CH), jnp.float32),
    )(x3, wb)

    def apply_body(stats_ref, x_ref, w_ref, g_ref, b_ref, o_ref):
        s = stats_ref[0:1, :]
        q = stats_ref[1:2, :]
        mean = s * (1.0 / N)
        var = q * (1.0 / N) - mean * mean
        inv = jax.lax.rsqrt(var + BN_EPS)
        scale = g_ref[...] * inv
        bias = b_ref[...] - mean * scale
        xb = x_ref[...].reshape(ROWSG2 * 8, IN_CH).astype(jnp.bfloat16)
        h = jnp.dot(xb, w_ref[...], preferred_element_type=jnp.float32)
        o = jnp.maximum(h * scale + bias, 0.0)
        o_ref[...] = o.reshape(ROWSG2, 8, OUT_CH)

    out3 = pl.pallas_call(
        apply_body,
        grid=(G // ROWSG2,),
        in_specs=[
            pl.BlockSpec((2, OUT_CH), lambda i: (0, 0)),
            pl.BlockSpec((ROWSG2, 8, IN_CH), lambda i: (i, 0, 0)),
            pl.BlockSpec((IN_CH, OUT_CH), lambda i: (0, 0)),
            pl.BlockSpec((1, OUT_CH), lambda i: (0, 0)),
            pl.BlockSpec((1, OUT_CH), lambda i: (0, 0)),
        ],
        out_specs=pl.BlockSpec((ROWSG2, 8, OUT_CH), lambda i: (i, 0, 0)),
        out_shape=jax.ShapeDtypeStruct((G, 8, OUT_CH), jnp.float32),
    )(stats, x3, wb, g2, b2)
    return out3.reshape(N, OUT_CH)
